# Initial kernel scaffold; baseline (speedup 1.0000x reference)
import jax
import jax.numpy as jnp
from jax.experimental import pallas as pl


def kernel(logits, labels):
    raise NotImplementedError("write your pallas kernel here")


# speedup vs baseline: 5.6928x; 5.6928x over previous
import functools

import jax
import jax.numpy as jnp
from jax.experimental import pallas as pl
from jax.experimental.pallas import tpu as pltpu

TOPK_FRAC = 0.2


def _fused_kernel(logits_ref, labels_ref, out_ref, keys_scratch, *, nblk, w, k,
                  nsteps):
    step = pl.program_id(0) * nblk + pl.program_id(1)

    x = logits_ref[0]
    lab = labels_ref[...].reshape(1, w)
    c = x.shape[0]
    m = jnp.max(x, axis=0, keepdims=True)
    s = jnp.sum(jnp.exp(x - m), axis=0, keepdims=True)
    lse = jnp.log(s) + m
    cls = jax.lax.broadcasted_iota(jnp.int32, (c, w), 0)
    picked = jnp.sum(jnp.where(cls == lab, x, 0.0), axis=0, keepdims=True)
    loss = lse - picked
    keys = jax.lax.bitcast_convert_type(loss, jnp.int32)
    rows = w // 1024
    keys_scratch[pl.ds(step * rows, rows), :] = keys.reshape(rows, 1024)

    @pl.when(step == nsteps - 1)
    def _select():
        kk = keys_scratch[...]

        def body(_, carry):
            lo, hi = carry
            mid = lo + ((hi - lo + 1) >> 1)
            cnt = jnp.sum((kk >= mid).astype(jnp.float32))
            pred = cnt >= k
            return (jnp.where(pred, mid, lo), jnp.where(pred, hi, mid - 1))

        lo, _ = jax.lax.fori_loop(0, 31, body, (jnp.int32(0),
                                                jnp.int32(0x7F7FFFFF)))
        vals = jax.lax.bitcast_convert_type(kk, jnp.float32)
        mask = kk >= lo
        cnt_t = jnp.sum(mask.astype(jnp.float32)).reshape(1, 1)
        sum_t = jnp.sum(jnp.where(mask, vals, 0.0)).reshape(1, 1)
        tval = jax.lax.bitcast_convert_type(
            jnp.full((1, 1), lo, jnp.int32), jnp.float32)
        out_ref[...] = (sum_t - (cnt_t - k) * tval) / k


@jax.jit
def kernel(logits, labels):
    b, c, h, wdim = logits.shape
    npix = h * wdim
    total = b * npix
    k = int(TOPK_FRAC * total)
    w = 16384
    nblk = npix // w
    nsteps = b * nblk
    rows_total = total // 1024

    logits3 = logits.reshape(b, c, npix)
    labels4 = labels.reshape(b, nblk, 1, w)

    out = pl.pallas_call(
        functools.partial(_fused_kernel, nblk=nblk, w=w, k=k, nsteps=nsteps),
        grid=(b, nblk),
        in_specs=[
            pl.BlockSpec((1, c, w), lambda i, j: (i, 0, j)),
            pl.BlockSpec((1, 1, 1, w), lambda i, j: (i, j, 0, 0)),
        ],
        out_specs=pl.BlockSpec((1, 1), lambda i, j: (0, 0)),
        out_shape=jax.ShapeDtypeStruct((1, 1), jnp.float32),
        scratch_shapes=[pltpu.VMEM((rows_total, 1024), jnp.int32)],
        compiler_params=pltpu.CompilerParams(
            dimension_semantics=("arbitrary", "arbitrary")),
    )(logits3, labels4)
    return out[0, 0]


# speedup vs baseline: 6.8981x; 1.2117x over previous
import functools

import jax
import jax.numpy as jnp
from jax.experimental import pallas as pl
from jax.experimental.pallas import tpu as pltpu

TOPK_FRAC = 0.2
_R = 8
_L = 2048


def _fused_kernel(logits_ref, labels_ref, out_ref, keys_scratch, *, nblk, k,
                  nsteps):
    step = pl.program_id(0) * nblk + pl.program_id(1)

    x = logits_ref[0, :, 0]
    lab = labels_ref[0, 0]
    c = x.shape[0]
    m = jnp.max(x, axis=0)
    s = jnp.sum(jnp.exp(x - m[None]), axis=0)
    lse = jnp.log(s) + m
    cls = jax.lax.broadcasted_iota(jnp.int32, (c, _R, _L), 0)
    picked = jnp.sum(jnp.where(cls == lab[None], x, 0.0), axis=0)
    loss = lse - picked
    keys = jax.lax.bitcast_convert_type(loss, jnp.int32)
    keys_scratch[pl.ds(step * _R, _R), :] = keys

    @pl.when(step == nsteps - 1)
    def _select():
        kk = keys_scratch[...]

        def body(_, carry):
            lo, hi = carry
            mid = lo + ((hi - lo + 1) >> 1)
            cnt = jnp.sum((kk >= mid).astype(jnp.float32))
            pred = cnt >= k
            return (jnp.where(pred, mid, lo), jnp.where(pred, hi, mid - 1))

        lo, _ = jax.lax.fori_loop(0, 31, body, (jnp.int32(0),
                                                jnp.int32(0x7F7FFFFF)))
        vals = jax.lax.bitcast_convert_type(kk, jnp.float32)
        mask = kk >= lo
        cnt_t = jnp.sum(mask.astype(jnp.float32)).reshape(1, 1)
        sum_t = jnp.sum(jnp.where(mask, vals, 0.0)).reshape(1, 1)
        tval = jax.lax.bitcast_convert_type(
            jnp.full((1, 1), lo, jnp.int32), jnp.float32)
        out_ref[...] = (sum_t - (cnt_t - k) * tval) / k


@jax.jit
def kernel(logits, labels):
    b, c, h, wdim = logits.shape
    npix = h * wdim
    total = b * npix
    k = int(TOPK_FRAC * total)
    w = _R * _L
    nblk = npix // w
    nsteps = b * nblk
    rows_total = total // _L

    logits5 = logits.reshape(b, c, nblk, _R, _L)
    labels4 = labels.reshape(b, nblk, _R, _L)

    out = pl.pallas_call(
        functools.partial(_fused_kernel, nblk=nblk, k=k, nsteps=nsteps),
        grid=(b, nblk),
        in_specs=[
            pl.BlockSpec((1, c, 1, _R, _L), lambda i, j: (i, 0, j, 0, 0)),
            pl.BlockSpec((1, 1, _R, _L), lambda i, j: (i, j, 0, 0)),
        ],
        out_specs=pl.BlockSpec((1, 1), lambda i, j: (0, 0)),
        out_shape=jax.ShapeDtypeStruct((1, 1), jnp.float32),
        scratch_shapes=[pltpu.VMEM((rows_total, _L), jnp.int32)],
        compiler_params=pltpu.CompilerParams(
            dimension_semantics=("arbitrary", "arbitrary")),
    )(logits5, labels4)
    return out[0, 0]
